# Initial kernel scaffold; baseline (speedup 1.0000x reference)
import jax
import jax.numpy as jnp
from jax.experimental import pallas as pl


def kernel(ray, ray_min, ray_max, grid):
    raise NotImplementedError("write your pallas kernel here")


# speedup vs baseline: 1.6847x; 1.6847x over previous
import functools

import jax
import jax.numpy as jnp
from jax import lax
from jax.experimental import pallas as pl
from jax.experimental.pallas import tpu as pltpu
from jax.experimental.pallas import tpu_sc as plsc

_LANES = 16
_NUM_CORES = 2
_NUM_SUBCORES = 16
_NUM_WORKERS = _NUM_CORES * _NUM_SUBCORES

_CHUNK = 800
_SUB = 80


def _lane_bcast(vec, j):
    sel = jnp.full((_LANES, 1), j, jnp.int32)
    dnums = lax.GatherDimensionNumbers(
        offset_dims=(), collapsed_slice_dims=(0,), start_index_map=(0,))
    return lax.gather(vec, sel, dnums, (1,),
                      mode=lax.GatherScatterMode.PROMISE_IN_BOUNDS)


def _transpose_body(g_ref, t_ref):
    t_ref[...] = g_ref[...].T


def _make_table(grid2d, blk):
    C, U = grid2d.shape
    nblk = (U + blk - 1) // blk
    return pl.pallas_call(
        _transpose_body,
        grid=(nblk,),
        in_specs=[pl.BlockSpec((C, blk), lambda j: (0, j))],
        out_specs=pl.BlockSpec((blk, C), lambda j: (j, 0)),
        out_shape=jax.ShapeDtypeStruct((U, C), jnp.float32),
    )(grid2d)


def _sc_lookup(ray1d, consts, table, chunk, sub):
    N = ray1d.shape[0]
    U, C = table.shape
    n_chunks = N // chunk
    n_iters = (n_chunks + _NUM_WORKERS - 1) // _NUM_WORKERS
    n_sub = chunk // sub
    umax = float(U - 1)

    mesh = plsc.VectorSubcoreMesh(
        core_axis_name="c", subcore_axis_name="s",
        num_cores=_NUM_CORES, num_subcores=_NUM_SUBCORES)

    @functools.partial(
        pl.kernel,
        mesh=mesh,
        compiler_params=pltpu.CompilerParams(use_tc_tiling_on_sc=False),
        out_type=jax.ShapeDtypeStruct((N, C), jnp.float32),
        scratch_types=[
            pltpu.VMEM((2, _LANES), jnp.float32),
            pltpu.VMEM((chunk,), jnp.float32),
            pltpu.VMEM((chunk,), jnp.int32),
            pltpu.VMEM((chunk,), jnp.int32),
            pltpu.VMEM((chunk,), jnp.float32),
            pltpu.VMEM((chunk,), jnp.float32),
            pltpu.VMEM((chunk, C), jnp.float32),
            pltpu.VMEM((chunk, C), jnp.float32),
            pltpu.SemaphoreType.DMA,
        ],
    )
    def body(ray_hbm, consts_hbm, table_hbm, out_hbm,
             consts_v, ray_v, idxb_v, idxt_v, omw_v, w_v, rows_b, rows_t, sem):
        wid = lax.axis_index("s") * _NUM_CORES + lax.axis_index("c")
        pltpu.sync_copy(consts_hbm, consts_v)
        rmin = consts_v[0, :]
        rinv = consts_v[1, :]

        def chunk_body(k, _):
            cid = wid + k * _NUM_WORKERS

            @pl.when(cid < n_chunks)
            def _():
                base = cid * chunk
                pltpu.sync_copy(ray_hbm.at[pl.ds(base, chunk)], ray_v)

                def lane_body(i, _):
                    off = i * _LANES
                    r = ray_v[pl.ds(off, _LANES)]
                    gi = ((r - rmin) * rinv) * umax
                    bt = gi.astype(jnp.int32)
                    bf = bt.astype(jnp.float32)
                    b = jnp.where(bf > gi, bt - 1, bt)
                    w = gi - b.astype(jnp.float32)
                    t = b + 1
                    validf = jnp.where((b >= 0) & (t <= U - 1),
                                       jnp.float32(1.0), jnp.float32(0.0))
                    idxb_v[pl.ds(off, _LANES)] = jnp.clip(b, 0, U - 1)
                    idxt_v[pl.ds(off, _LANES)] = jnp.clip(t, 0, U - 1)
                    omw_v[pl.ds(off, _LANES)] = (1.0 - w) * validf
                    w_v[pl.ds(off, _LANES)] = w * validf
                    return 0

                lax.fori_loop(0, chunk // _LANES, lane_body, 0)

                handles = []
                for j in range(n_sub):
                    sl = pl.ds(j * sub, sub)
                    handles.append(pltpu.async_copy(
                        table_hbm.at[idxb_v.at[sl]], rows_b.at[sl], sem))
                    handles.append(pltpu.async_copy(
                        table_hbm.at[idxt_v.at[sl]], rows_t.at[sl], sem))
                for h in handles:
                    h.wait()

                def blend_body(i, _):
                    off = i * _LANES
                    omw16 = omw_v[pl.ds(off, _LANES)]
                    w16 = w_v[pl.ds(off, _LANES)]
                    for j in range(_LANES):
                        omwj = _lane_bcast(omw16, j)
                        wj = _lane_bcast(w16, j)
                        row = off + j
                        lo = pl.ds(0, _LANES)
                        hi = pl.ds(_LANES, _LANES)
                        rows_b[row, lo] = rows_b[row, lo] * omwj + rows_t[row, lo] * wj
                        rows_b[row, hi] = rows_b[row, hi] * omwj + rows_t[row, hi] * wj
                    return 0

                lax.fori_loop(0, chunk // _LANES, blend_body, 0)

                pltpu.sync_copy(rows_b, out_hbm.at[pl.ds(base, chunk)])

            return 0

        lax.fori_loop(0, n_iters, chunk_body, 0)

    return body(ray1d, consts, table)


def kernel(ray, ray_min, ray_max, grid):
    N = ray.shape[0]
    C = grid.shape[1]
    U = grid.shape[-1]
    assert N % _CHUNK == 0 and _CHUNK % _SUB == 0 and _SUB % 8 == 0
    assert C == 2 * _LANES

    grid2d = grid.reshape(C, U)
    table = _make_table(grid2d, 4096)

    ray1d = ray.reshape(N)
    rinv = 1.0 / (ray_max - ray_min)
    consts = jnp.stack([
        jnp.broadcast_to(ray_min.astype(jnp.float32), (_LANES,)),
        jnp.broadcast_to(rinv.astype(jnp.float32), (_LANES,)),
    ])
    return _sc_lookup(ray1d, consts, table, _CHUNK, _SUB)
